# Initial kernel scaffold; baseline (speedup 1.0000x reference)
import jax
import jax.numpy as jnp
from jax.experimental import pallas as pl


def kernel(x, edge_index, edge_attr, Wq, Wk, Wv, We, att_q, att_k, att_e, bias, bn_gamma, bn_beta, W_out, b_out):
    raise NotImplementedError("write your pallas kernel here")


# speedup vs baseline: 1.1647x; 1.1647x over previous
import functools

import jax
import jax.numpy as jnp
from jax.experimental import pallas as pl
from jax.experimental.pallas import tpu as pltpu

N = 10000
E = 320000
D_IN = 128
H = 8
C = 16
HC = 128
D_EDGE = 16
L = 3
D_OUT = 128

_BN = 1000


def _proj_body(x_ref, wqs_ref, wk_ref, wv_ref, qa_ref, kv_ref):
    x = x_ref[...]
    qa_ref[...] = jnp.dot(x, wqs_ref[...], preferred_element_type=jnp.float32)
    kv_ref[:, :HC] = jnp.dot(x, wk_ref[...], preferred_element_type=jnp.float32)
    kv_ref[:, HC:] = jnp.dot(x, wv_ref[...], preferred_element_type=jnp.float32)


def _proj(h, wqs, wkT, wvT):
    grid = (N // _BN,)
    return pl.pallas_call(
        _proj_body,
        grid=grid,
        in_specs=[
            pl.BlockSpec((_BN, D_IN), lambda i: (i, 0)),
            pl.BlockSpec((D_IN, HC), lambda i: (0, 0)),
            pl.BlockSpec((D_IN, HC), lambda i: (0, 0)),
            pl.BlockSpec((D_IN, HC), lambda i: (0, 0)),
        ],
        out_specs=[
            pl.BlockSpec((_BN, HC), lambda i: (i, 0)),
            pl.BlockSpec((_BN, 2 * HC), lambda i: (i, 0)),
        ],
        out_shape=[
            jax.ShapeDtypeStruct((N, HC), jnp.float32),
            jax.ShapeDtypeStruct((N, 2 * HC), jnp.float32),
        ],
    )(h, wqs, wkT, wvT)


def kernel(x, edge_index, edge_attr, Wq, Wk, Wv, We, att_q, att_k, att_e,
           bias, bn_gamma, bn_beta, W_out, b_out):
    src = edge_index[0]
    dst = edge_index[1]

    sqk = (att_q * att_k).reshape(L, HC)
    WqsT = Wq.transpose(0, 2, 1) * sqk[:, None, :]
    WkT = Wk.transpose(0, 2, 1)
    WvT = Wv.transpose(0, 2, 1)
    M = jnp.einsum("lhcd,lhc->ldh", We.reshape(L, H, C, D_EDGE), att_e)
    ealpha_all = jnp.einsum("ed,ldh->leh", edge_attr, M)

    h = x
    for l in range(L):
        qa, kv = _proj(h, WqsT[l], WkT[l], WvT[l])
        k = kv[:, :HC]
        v = kv[:, HC:]

        qd = qa[dst].reshape(E, H, C)
        ks = k[src].reshape(E, H, C)
        alpha = (qd * ks).sum(-1) + ealpha_all[l]
        alpha = jax.nn.leaky_relu(alpha, negative_slope=0.2)
        ex = jnp.exp(alpha)
        denom = jax.ops.segment_sum(ex, dst, num_segments=N)
        vnum = jax.ops.segment_sum(
            v[src].reshape(E, H, C) * ex[:, :, None], dst, num_segments=N)
        out = vnum / (denom[:, :, None] + 1e-16)
        hh = out.reshape(N, HC) + bias[l]

        mu = hh.mean(axis=0)
        var = hh.var(axis=0)
        hh = bn_gamma[l] * (hh - mu) / jnp.sqrt(var + 1e-5) + bn_beta[l]
        h = jax.nn.elu(hh)

    return h @ W_out.T + b_out


# speedup vs baseline: 13.1347x; 11.2771x over previous
import functools

import jax
import jax.numpy as jnp
from jax import lax
from jax.experimental import pallas as pl
from jax.experimental.pallas import tpu as pltpu
from jax.experimental.pallas import tpu_sc as plsc

N = 10000
E = 320000
D_IN = 128
H = 8
C = 16
HC = 128
D_EDGE = 16
L = 3
D_OUT = 128

_NC = 2
_NS = 16
_NW = _NC * _NS
_PW = E // _NW
_B = 80
_NB = _PW // _B
_AW = 136
_NPAD = 10240
_NPS = _NPAD // _NS

_BN = 1000
_GN = N // _BN
_BE = 3200
_LH = L * H


def _ea_body(m_ref, ea_ref, out_ref):
    out_ref[0] = jnp.dot(ea_ref[...], m_ref[0],
                         preferred_element_type=jnp.float32)


def _ealpha(M, edge_attr):
    return pl.pallas_call(
        _ea_body,
        grid=(L, E // _BE),
        in_specs=[
            pl.BlockSpec((1, D_EDGE, H), lambda l, i: (l, 0, 0)),
            pl.BlockSpec((_BE, D_EDGE), lambda l, i: (i, 0)),
        ],
        out_specs=pl.BlockSpec((1, _BE, H), lambda l, i: (l, i, 0)),
        out_shape=jax.ShapeDtypeStruct((L, E, H), jnp.float32),
    )(M, edge_attr)


def _bn_elu(h, st, gam, bet):
    mu = st[0:1] / N
    var = st[1:2] / N - mu * mu
    inv = lax.rsqrt(var + 1e-5)
    xn = (h - mu) * (gam * inv) + bet
    return jnp.where(xn > 0, xn, jnp.exp(xn) - 1.0)


def _proj_body(with_bn, x_ref, st_ref, gam_ref, bet_ref, wqs_ref, wk_ref,
               wv_ref, qa_ref, kv_ref):
    x = x_ref[...]
    if with_bn:
        x = _bn_elu(x, st_ref[...], gam_ref[...], bet_ref[...])
    qa_ref[...] = jnp.dot(x, wqs_ref[...], preferred_element_type=jnp.float32)
    kv_ref[:, :HC] = jnp.dot(x, wk_ref[...], preferred_element_type=jnp.float32)
    kv_ref[:, HC:] = jnp.dot(x, wv_ref[...], preferred_element_type=jnp.float32)


def _proj(h, st, gam, bet, wqs, wkT, wvT, with_bn):
    return pl.pallas_call(
        functools.partial(_proj_body, with_bn),
        grid=(_GN,),
        in_specs=[
            pl.BlockSpec((_BN, D_IN), lambda i: (i, 0)),
            pl.BlockSpec((2, HC), lambda i: (0, 0)),
            pl.BlockSpec((1, HC), lambda i: (0, 0)),
            pl.BlockSpec((1, HC), lambda i: (0, 0)),
            pl.BlockSpec((D_IN, HC), lambda i: (0, 0)),
            pl.BlockSpec((D_IN, HC), lambda i: (0, 0)),
            pl.BlockSpec((D_IN, HC), lambda i: (0, 0)),
        ],
        out_specs=[
            pl.BlockSpec((_BN, HC), lambda i: (i, 0)),
            pl.BlockSpec((_BN, 2 * HC), lambda i: (i, 0)),
        ],
        out_shape=[
            jax.ShapeDtypeStruct((N, HC), jnp.float32),
            jax.ShapeDtypeStruct((N, 2 * HC), jnp.float32),
        ],
    )(h, st, gam, bet, wqs, wkT, wvT)


def _post_body(a_ref, bias_ref, hh_ref, st_ref):
    i = pl.program_id(0)
    a = a_ref[0] + a_ref[1]
    den = a[:, 0:H]
    vs = a[:, H:H + HC].reshape(_BN, H, C)
    hh = (vs / (den[:, :, None] + 1e-16)).reshape(_BN, HC) + bias_ref[...]
    hh_ref[...] = hh
    s = jnp.sum(hh, axis=0, keepdims=True)
    s2 = jnp.sum(hh * hh, axis=0, keepdims=True)
    st = jnp.concatenate([s, s2], axis=0)

    @pl.when(i == 0)
    def _():
        st_ref[...] = st

    @pl.when(i > 0)
    def _():
        st_ref[...] += st


def _post(acc, bias_l):
    return pl.pallas_call(
        _post_body,
        grid=(_GN,),
        in_specs=[
            pl.BlockSpec((_NC, _BN, _AW), lambda i: (0, i, 0)),
            pl.BlockSpec((1, HC), lambda i: (0, 0)),
        ],
        out_specs=[
            pl.BlockSpec((_BN, HC), lambda i: (i, 0)),
            pl.BlockSpec((2, HC), lambda i: (0, 0)),
        ],
        out_shape=[
            jax.ShapeDtypeStruct((N, HC), jnp.float32),
            jax.ShapeDtypeStruct((2, HC), jnp.float32),
        ],
    )(acc, bias_l)


def _final_body(x_ref, st_ref, gam_ref, bet_ref, w_ref, b_ref, o_ref):
    x = _bn_elu(x_ref[...], st_ref[...], gam_ref[...], bet_ref[...])
    o_ref[...] = jnp.dot(x, w_ref[...],
                         preferred_element_type=jnp.float32) + b_ref[...]


def _final(h, st, gam, bet, woT, b):
    return pl.pallas_call(
        _final_body,
        grid=(_GN,),
        in_specs=[
            pl.BlockSpec((_BN, HC), lambda i: (i, 0)),
            pl.BlockSpec((2, HC), lambda i: (0, 0)),
            pl.BlockSpec((1, HC), lambda i: (0, 0)),
            pl.BlockSpec((1, HC), lambda i: (0, 0)),
            pl.BlockSpec((HC, D_OUT), lambda i: (0, 0)),
            pl.BlockSpec((1, D_OUT), lambda i: (0, 0)),
        ],
        out_specs=pl.BlockSpec((_BN, D_OUT), lambda i: (i, 0)),
        out_shape=jax.ShapeDtypeStruct((N, D_OUT), jnp.float32),
    )(h, st, gam, bet, woT, b)


def _edge_body(qa_hbm, kv_hbm, eaT_hbm, src_hbm, dst_hbm, out_hbm,
               src_v, dst_v, ea_v, qrows, kvrows, wv, acc,
               sem_q, sem_kv):
    core = lax.axis_index("c")
    sid = lax.axis_index("s")
    zv = jnp.zeros((16,), jnp.float32)
    iota = lax.iota(jnp.int32, 16)

    def _zrow(i, carry):
        for j in range(_AW // 16):
            wv[i, pl.ds(j * 16, 16)] = zv
        wv[i, pl.ds(_AW - 16, 16)] = zv
        return carry

    lax.fori_loop(0, _B, _zrow, 0)
    for t in range(_NPS // _B):
        pltpu.sync_copy(wv, acc.at[pl.ds(sid * _NPS + t * _B, _B)])
    plsc.subcore_barrier()

    ebase = (core * _NS + sid) * _PW

    def _block(bi, carry):
        eb = ebase + bi * _B
        pltpu.sync_copy(src_hbm.at[pl.ds(eb, _B)], src_v)
        pltpu.sync_copy(dst_hbm.at[pl.ds(eb, _B)], dst_v)
        pltpu.sync_copy(eaT_hbm.at[pl.ds(eb, _B)], ea_v)
        cq = pltpu.async_copy(qa_hbm.at[dst_v], qrows, sem_q)
        ckv = pltpu.async_copy(kv_hbm.at[src_v], kvrows, sem_kv)
        cq.wait()
        ckv.wait()

        def _group(g, gcarry):
            row = g * 16 + iota
            for h in range(H):
                a = plsc.load_gather(ea_v, [row, jnp.full((16,), h, jnp.int32)])
                for c in range(C):
                    col = jnp.full((16,), h * C + c, jnp.int32)
                    qc = plsc.load_gather(qrows, [row, col])
                    kc = plsc.load_gather(kvrows, [row, col])
                    a = a + qc * kc
                a = jnp.maximum(a, 0.2 * a)
                ex = jnp.exp(a)
                plsc.store_scatter(wv, [row, jnp.full((16,), h, jnp.int32)], ex)
                for c in range(C):
                    vcol = jnp.full((16,), HC + h * C + c, jnp.int32)
                    vc = plsc.load_gather(kvrows, [row, vcol])
                    ocol = jnp.full((16,), H + h * C + c, jnp.int32)
                    plsc.store_scatter(wv, [row, ocol], ex * vc)
            return gcarry

        lax.fori_loop(0, _B // 16, _group, 0)
        pltpu.sync_copy(wv, acc.at[dst_v], add=True)
        return carry

    lax.fori_loop(0, _NB, _block, 0)
    plsc.subcore_barrier()
    for t in range(_NPS // _B):
        r0 = sid * _NPS + t * _B
        pltpu.sync_copy(acc.at[pl.ds(r0, _B)], out_hbm.at[core, pl.ds(r0, _B)])


@functools.partial(
    pl.kernel,
    out_type=jax.ShapeDtypeStruct((_NC, _NPAD, _AW), jnp.float32),
    mesh=plsc.VectorSubcoreMesh(core_axis_name="c", subcore_axis_name="s"),
    compiler_params=pltpu.CompilerParams(
        use_tc_tiling_on_sc=False, needs_layout_passes=False),
    scratch_types=[
        pltpu.VMEM((_B,), jnp.int32),
        pltpu.VMEM((_B,), jnp.int32),
        pltpu.VMEM((_B, H), jnp.float32),
        pltpu.VMEM((_B, HC), jnp.float32),
        pltpu.VMEM((_B, 2 * HC), jnp.float32),
        pltpu.VMEM((_B, _AW), jnp.float32),
        pltpu.VMEM_SHARED((_NPAD, _AW), jnp.float32),
        pltpu.SemaphoreType.DMA,
        pltpu.SemaphoreType.DMA,
    ],
)
def _edge_kernel(qa_hbm, kv_hbm, eaT_hbm, src_hbm, dst_hbm, out_hbm,
                 src_v, dst_v, ea_v, qrows, kvrows, wv, acc,
                 sem_q, sem_kv):
    _edge_body(qa_hbm, kv_hbm, eaT_hbm, src_hbm, dst_hbm, out_hbm,
               src_v, dst_v, ea_v, qrows, kvrows, wv, acc,
               sem_q, sem_kv)


def kernel(x, edge_index, edge_attr, Wq, Wk, Wv, We, att_q, att_k, att_e,
           bias, bn_gamma, bn_beta, W_out, b_out):
    src = edge_index[0]
    dst = edge_index[1]

    sqk = (att_q * att_k).reshape(L, HC)
    WqsT = Wq.transpose(0, 2, 1) * sqk[:, None, :]
    WkT = Wk.transpose(0, 2, 1)
    WvT = Wv.transpose(0, 2, 1)
    M = jnp.einsum("lhcd,lhc->ldh", We.reshape(L, H, C, D_EDGE), att_e)

    ea_all = _ealpha(M, edge_attr)

    bias2 = bias.reshape(L, 1, HC)
    gam2 = bn_gamma.reshape(L, 1, HC)
    bet2 = bn_beta.reshape(L, 1, HC)
    dummy_st = jnp.zeros((2, HC), jnp.float32)

    h = x
    st = dummy_st
    for l in range(L):
        qa, kv = _proj(h, st, gam2[max(l - 1, 0)], bet2[max(l - 1, 0)],
                       WqsT[l], WkT[l], WvT[l], with_bn=(l > 0))
        acc = _edge_kernel(qa, kv, ea_all[l], src, dst)
        h, st = _post(acc, bias2[l])

    return _final(h, st, gam2[L - 1], bet2[L - 1], W_out.T, b_out.reshape(1, D_OUT))
